# baseline (speedup 1.0000x reference)
import math

import jax
import jax.numpy as jnp
from jax.experimental import pallas as pl
from jax.experimental.pallas import tpu as pltpu

_B, _N, _C = 2, 2048, 128
_CHUNK = 512
_KC = 128
_KF = 16
_TEMP = 0.07
_COS_T = math.cos(math.radians(15.0))
_INT_MIN = -2147483648
_INT_MAX = 2147483647


def _count_ge(sk, mid, ones8):
    ind = jnp.where(sk >= mid, jnp.float32(1), jnp.float32(0))
    return jnp.dot(ind.astype(jnp.bfloat16), ones8,
                   preferred_element_type=jnp.float32)[:, 0:1]


def _kth_largest_key(sk, k):
    ones8 = jnp.full((sk.shape[1], 8), jnp.bfloat16(1))
    kf = jnp.float32(k)
    cnt0 = _count_ge(sk, jnp.int32(0), ones8)
    ok0 = cnt0 >= kf
    lo = jnp.where(ok0, jnp.int32(0), jnp.int32(_INT_MIN))
    hi = jnp.where(ok0, jnp.int32(_INT_MAX), jnp.int32(-1))

    def body(_, carry):
        lo, hi = carry
        d = hi - lo
        mid = lo + (d >> 1) + (d & 1)
        ok = _count_ge(sk, mid, ones8) >= kf
        return jnp.where(ok, mid, lo), jnp.where(ok, hi, mid - 1)

    lo, hi = jax.lax.fori_loop(0, 31, body, (lo, hi))
    return lo


def _kth_largest_masked(skm, k):
    def body(_, cur):
        vals = jnp.where(skm < cur, skm, jnp.int32(_INT_MIN))
        return jnp.max(vals, axis=1, keepdims=True)

    return jax.lax.fori_loop(0, k, body,
                             jnp.full((skm.shape[0], 1), _INT_MAX, jnp.int32))


def _ikey(x):
    xb = jax.lax.bitcast_convert_type(x, jnp.int32)
    return jnp.where(xb >= 0, xb, jnp.int32(_INT_MIN) - xb)


def _body(fa_ref, fbt_ref, dap_ref, dbtp_ref, dbp_ref, wqt_ref, bq_ref,
          wk_ref, bk_ref, out_ref, kth_ref, ktl_ref):
    c = pl.program_id(1)

    @pl.when(c == 0)
    def _():
        kt = (
            jnp.dot(wk_ref[...].astype(jnp.bfloat16),
                    fbt_ref[0].astype(jnp.bfloat16),
                    preferred_element_type=jnp.float32)
            + bk_ref[:, 0:1]
        )
        kth = kt.astype(jnp.bfloat16)
        kth_ref[...] = kth
        ktl_ref[...] = (kt - kth.astype(jnp.float32)).astype(jnp.bfloat16)

    q = (jnp.dot(fa_ref[0].astype(jnp.bfloat16),
                 wqt_ref[...].astype(jnp.bfloat16),
                 preferred_element_type=jnp.float32)
         + bq_ref[0:1, :])
    q_hi = q.astype(jnp.bfloat16)
    q_lo = (q - q_hi.astype(jnp.float32)).astype(jnp.bfloat16)
    kth = kth_ref[...]
    sim_sel = jnp.dot(q_hi, kth, preferred_element_type=jnp.float32)
    simf = (sim_sel
            + jnp.dot(q_hi, ktl_ref[...], preferred_element_type=jnp.float32)
            + jnp.dot(q_lo, kth, preferred_element_type=jnp.float32))
    da = dap_ref[0]
    dbt = dbtp_ref[0]
    cos = (da[:, 0:1] * dbt[0:1, :] + da[:, 1:2] * dbt[1:2, :]
           + da[:, 2:3] * dbt[2:3, :])
    ang_ok = cos >= jnp.float32(_COS_T)

    sk = _ikey(sim_sel)
    skf = _ikey(simf)

    t128 = _kth_largest_key(sk, _KC)
    cand = sk >= t128
    lm = jnp.logical_and(cand, ang_ok)
    nvalid = jnp.sum(jnp.where(lm, jnp.int32(1), jnp.int32(0)),
                     axis=1, keepdims=True)
    no_vm = nvalid == 0
    elig = jnp.logical_or(lm, jnp.logical_and(cand, no_vm))
    skm = jnp.where(elig, skf, jnp.int32(_INT_MIN))
    t16 = _kth_largest_masked(skm, _KF)
    sel = jnp.logical_and(elig, skf >= t16)

    m = jnp.max(jnp.where(sel, simf, -jnp.inf), axis=1, keepdims=True)
    w = jnp.where(sel, jnp.exp((simf - m) / jnp.float32(_TEMP)), 0.0)
    z = jnp.sum(w, axis=1, keepdims=True)
    wn = (w / z).astype(jnp.bfloat16)
    wx = jnp.dot(wn, dbp_ref[0].astype(jnp.bfloat16),
                 preferred_element_type=jnp.float32)

    rows = [jnp.sum(da[:, i:i + 1] * wx, axis=0).reshape(1, 8)
            for i in range(3)]
    upd = jnp.concatenate(rows + [jnp.zeros((5, 8), jnp.float32)], axis=0)

    @pl.when(c == 0)
    def _():
        out_ref[...] = jnp.zeros((1, 8, 8), jnp.float32)

    out_ref[...] += upd[None]


def kernel(feat_a, feat_b, dirs_a, dirs_b, Wq, bq, Wk, bk):
    fbt = jnp.swapaxes(feat_b, 1, 2)
    wqt = Wq.T
    dap = jnp.pad(dirs_a, ((0, 0), (0, 0), (0, 5)))
    dbp = jnp.pad(dirs_b, ((0, 0), (0, 0), (0, 5)))
    dbtp = jnp.swapaxes(dbp, 1, 2)
    bq_in = jnp.tile(bq.reshape(1, _C), (8, 1))
    bk_in = jnp.tile(bk.reshape(_C, 1), (1, _C))

    hp = pl.pallas_call(
        _body,
        grid=(_B, _N // _CHUNK),
        in_specs=[
            pl.BlockSpec((1, _CHUNK, _C), lambda b, c: (b, c, 0)),
            pl.BlockSpec((1, _C, _N), lambda b, c: (b, 0, 0)),
            pl.BlockSpec((1, _CHUNK, 8), lambda b, c: (b, c, 0)),
            pl.BlockSpec((1, 8, _N), lambda b, c: (b, 0, 0)),
            pl.BlockSpec((1, _N, 8), lambda b, c: (b, 0, 0)),
            pl.BlockSpec((_C, _C), lambda b, c: (0, 0)),
            pl.BlockSpec((8, _C), lambda b, c: (0, 0)),
            pl.BlockSpec((_C, _C), lambda b, c: (0, 0)),
            pl.BlockSpec((_C, _C), lambda b, c: (0, 0)),
        ],
        out_specs=pl.BlockSpec((1, 8, 8), lambda b, c: (b, 0, 0)),
        out_shape=jax.ShapeDtypeStruct((_B, 8, 8), jnp.float32),
        scratch_shapes=[pltpu.VMEM((_C, _N), jnp.bfloat16),
                        pltpu.VMEM((_C, _N), jnp.bfloat16)],
    )(feat_a, fbt, dap, dbtp, dbp, wqt, bq_in, Wk, bk_in)

    H = hp[:, :3, :3]
    U, _, Vh = jnp.linalg.svd(H, full_matrices=False)
    R = jnp.matmul(Vh, jnp.swapaxes(U, -2, -1))
    det = jnp.linalg.det(R)
    sign = jnp.where(det < 0, -1.0, 1.0)
    Vh = Vh.at[:, :, -1].multiply(sign[:, None])
    R = jnp.matmul(Vh, jnp.swapaxes(U, -2, -1))
    return R


# speedup vs baseline: 1.1216x; 1.1216x over previous
import math

import jax
import jax.numpy as jnp
from jax.experimental import pallas as pl
from jax.experimental.pallas import tpu as pltpu

_B, _N, _C = 2, 2048, 128
_CHUNK = 1024
_KC = 128
_KF = 16
_TEMP = 0.07
_COS_T = math.cos(math.radians(15.0))
_INT_MIN = -2147483648
_INT_MAX = 2147483647


def _count_ge(sk, mid, emask):
    cond = sk >= mid
    if emask is not None:
        cond = jnp.logical_and(cond, emask)
    ones = jnp.where(cond, jnp.int32(1), jnp.int32(0))
    return jnp.sum(ones, axis=1, keepdims=True)


def _kth_largest_key(sk, k, emask):
    cnt0 = _count_ge(sk, jnp.int32(0), emask)
    ok0 = cnt0 >= k
    lo = jnp.where(ok0, jnp.int32(0), jnp.int32(_INT_MIN))
    hi = jnp.where(ok0, jnp.int32(_INT_MAX), jnp.int32(-1))

    def body(_, carry):
        lo, hi = carry
        d = hi - lo
        mid = lo + (d >> 1) + (d & 1)
        ok = _count_ge(sk, mid, emask) >= k
        return jnp.where(ok, mid, lo), jnp.where(ok, hi, mid - 1)

    lo, hi = jax.lax.fori_loop(0, 31, body, (lo, hi))
    return lo


def _kth_largest_masked(skm, k):
    def body(_, cur):
        vals = jnp.where(skm < cur, skm, jnp.int32(_INT_MIN))
        return jnp.max(vals, axis=1, keepdims=True)

    return jax.lax.fori_loop(0, k, body,
                             jnp.full((skm.shape[0], 1), _INT_MAX, jnp.int32))


def _ikey(x):
    xb = jax.lax.bitcast_convert_type(x, jnp.int32)
    return jnp.where(xb >= 0, xb, jnp.int32(_INT_MIN) - xb)


def _body(fa_ref, fbt_ref, dap_ref, dbtp_ref, dbp_ref, wqt_ref, bq_ref,
          wk_ref, bk_ref, out_ref, kth_ref, ktl_ref):
    c = pl.program_id(1)

    @pl.when(c == 0)
    def _():
        kt = (
            jnp.dot(wk_ref[...].astype(jnp.bfloat16),
                    fbt_ref[0].astype(jnp.bfloat16),
                    preferred_element_type=jnp.float32)
            + bk_ref[:, 0:1]
        )
        kth = kt.astype(jnp.bfloat16)
        kth_ref[...] = kth
        ktl_ref[...] = (kt - kth.astype(jnp.float32)).astype(jnp.bfloat16)

    q = (jnp.dot(fa_ref[0].astype(jnp.bfloat16),
                 wqt_ref[...].astype(jnp.bfloat16),
                 preferred_element_type=jnp.float32)
         + bq_ref[0:1, :])
    q_hi = q.astype(jnp.bfloat16)
    q_lo = (q - q_hi.astype(jnp.float32)).astype(jnp.bfloat16)
    kth = kth_ref[...]
    sim_sel = jnp.dot(q_hi, kth, preferred_element_type=jnp.float32)
    simf = (sim_sel
            + jnp.dot(q_hi, ktl_ref[...], preferred_element_type=jnp.float32)
            + jnp.dot(q_lo, kth, preferred_element_type=jnp.float32))
    da = dap_ref[0]
    dbt = dbtp_ref[0]
    cos = (da[:, 0:1] * dbt[0:1, :] + da[:, 1:2] * dbt[1:2, :]
           + da[:, 2:3] * dbt[2:3, :])
    ang_ok = cos >= jnp.float32(_COS_T)

    sk = _ikey(sim_sel)
    skf = _ikey(simf)

    t128 = _kth_largest_key(sk, _KC, None)
    cand = sk >= t128
    lm = jnp.logical_and(cand, ang_ok)
    nvalid = jnp.sum(jnp.where(lm, jnp.int32(1), jnp.int32(0)),
                     axis=1, keepdims=True)
    no_vm = nvalid == 0
    elig = jnp.logical_or(lm, jnp.logical_and(cand, no_vm))
    skm = jnp.where(elig, skf, jnp.int32(_INT_MIN))
    t16 = _kth_largest_masked(skm, _KF)
    sel = jnp.logical_and(elig, skf >= t16)

    m = jnp.max(jnp.where(sel, simf, -jnp.inf), axis=1, keepdims=True)
    w = jnp.where(sel, jnp.exp((simf - m) / jnp.float32(_TEMP)), 0.0)
    z = jnp.sum(w, axis=1, keepdims=True)
    wn = (w / z).astype(jnp.bfloat16)
    wx = jnp.dot(wn, dbp_ref[0].astype(jnp.bfloat16),
                 preferred_element_type=jnp.float32)

    rows = [jnp.sum(da[:, i:i + 1] * wx, axis=0).reshape(1, 8)
            for i in range(3)]
    upd = jnp.concatenate(rows + [jnp.zeros((5, 8), jnp.float32)], axis=0)

    @pl.when(c == 0)
    def _():
        out_ref[...] = jnp.zeros((1, 8, 8), jnp.float32)

    out_ref[...] += upd[None]


def kernel(feat_a, feat_b, dirs_a, dirs_b, Wq, bq, Wk, bk):
    fbt = jnp.swapaxes(feat_b, 1, 2)
    wqt = Wq.T
    dap = jnp.pad(dirs_a, ((0, 0), (0, 0), (0, 5)))
    dbp = jnp.pad(dirs_b, ((0, 0), (0, 0), (0, 5)))
    dbtp = jnp.swapaxes(dbp, 1, 2)
    bq_in = jnp.tile(bq.reshape(1, _C), (8, 1))
    bk_in = jnp.tile(bk.reshape(_C, 1), (1, _C))

    hp = pl.pallas_call(
        _body,
        grid=(_B, _N // _CHUNK),
        in_specs=[
            pl.BlockSpec((1, _CHUNK, _C), lambda b, c: (b, c, 0)),
            pl.BlockSpec((1, _C, _N), lambda b, c: (b, 0, 0)),
            pl.BlockSpec((1, _CHUNK, 8), lambda b, c: (b, c, 0)),
            pl.BlockSpec((1, 8, _N), lambda b, c: (b, 0, 0)),
            pl.BlockSpec((1, _N, 8), lambda b, c: (b, 0, 0)),
            pl.BlockSpec((_C, _C), lambda b, c: (0, 0)),
            pl.BlockSpec((8, _C), lambda b, c: (0, 0)),
            pl.BlockSpec((_C, _C), lambda b, c: (0, 0)),
            pl.BlockSpec((_C, _C), lambda b, c: (0, 0)),
        ],
        out_specs=pl.BlockSpec((1, 8, 8), lambda b, c: (b, 0, 0)),
        out_shape=jax.ShapeDtypeStruct((_B, 8, 8), jnp.float32),
        scratch_shapes=[pltpu.VMEM((_C, _N), jnp.bfloat16),
                        pltpu.VMEM((_C, _N), jnp.bfloat16)],
    )(feat_a, fbt, dap, dbtp, dbp, wqt, bq_in, Wk, bk_in)

    H = hp[:, :3, :3]
    U, _, Vh = jnp.linalg.svd(H, full_matrices=False)
    R = jnp.matmul(Vh, jnp.swapaxes(U, -2, -1))
    det = jnp.linalg.det(R)
    sign = jnp.where(det < 0, -1.0, 1.0)
    Vh = Vh.at[:, :, -1].multiply(sign[:, None])
    R = jnp.matmul(Vh, jnp.swapaxes(U, -2, -1))
    return R
